# baseline (speedup 1.0000x reference)
import functools

import jax
import jax.numpy as jnp
from jax import lax
from jax.experimental import pallas as pl
from jax.experimental.pallas import tpu as pltpu
from jax.experimental.pallas import tpu_sc as plsc

B = 16384
D = 64
QL = 20
DL = 50
NW = 32
BW = B // NW
SB = 1
QT = SB * QL
DT = SB * DL
NSTEP = BW // SB
NB = 4
LANES = 16
NCH = D // LANES

QEPS = float(QL * 1e-12)
DEPS = float(DL * 1e-12)


def _sqrt16(x):
    m = x
    s = jnp.full((LANES,), 1.0, jnp.float32)
    for k in (32, 16, 8, 4, 2, 1):
        c = m >= jnp.float32(4.0 ** k)
        m = jnp.where(c, m * jnp.float32(4.0 ** -k), m)
        s = jnp.where(c, s * jnp.float32(2.0 ** -k), s)
    for k in (32, 16, 8, 4, 2, 1):
        c = m < jnp.float32(4.0 ** -k)
        m = jnp.where(c, m * jnp.float32(4.0 ** k), m)
        s = jnp.where(c, s * jnp.float32(2.0 ** k), s)
    c = m >= jnp.float32(1.0)
    m = jnp.where(c, m * jnp.float32(0.25), m)
    s = jnp.where(c, s * jnp.float32(0.5), s)
    r = jnp.float32(7.0 / 3.0) - jnp.float32(4.0 / 3.0) * m
    for _ in range(5):
        r = r * (jnp.float32(1.5) - jnp.float32(0.5) * m * r * r)
    return jnp.where(x > 0, x * r * s, jnp.float32(0.0))


def _dssm_body(qs_ref, ds_ref, qt_ref, dt_ref, out_ref,
               qidx, didx, qb0, qb1, qb2, qb3, db0, db1, db2, db3,
               dots, nqs, nds, sims,
               qs0, qs1, qs2, qs3, ds0, ds1, ds2, ds3):
    wid = lax.axis_index("s") * 2 + lax.axis_index("c")
    pltpu.sync_copy(qs_ref.at[pl.ds(wid * BW, BW)], qidx)
    pltpu.sync_copy(ds_ref.at[pl.ds(wid * BW, BW)], didx)
    qbufs, dbufs = (qb0, qb1, qb2, qb3), (db0, db1, db2, db3)
    qsems, dsems = (qs0, qs1, qs2, qs3), (ds0, ds1, ds2, ds3)

    def issue(t, b):
        pltpu.async_copy(qt_ref.at[qidx.at[t]], qbufs[b], qsems[b])
        pltpu.async_copy(dt_ref.at[didx.at[t]], dbufs[b], dsems[b])

    def wait(b):
        pltpu.make_async_copy(qt_ref.at[pl.ds(0, QT)], qbufs[b], qsems[b]).wait()
        pltpu.make_async_copy(dt_ref.at[pl.ds(0, DT)], dbufs[b], dsems[b]).wait()

    def tok_sum(buf, base, n, unroll):
        def body(k, acc):
            return tuple(acc[c] + buf[base + k, pl.ds(c * LANES, LANES)]
                         for c in range(NCH))
        return lax.fori_loop(0, n, body,
                             tuple(jnp.zeros((LANES,), jnp.float32)
                                   for _ in range(NCH)),
                             unroll=unroll)

    lane_iota = jnp.arange(LANES, dtype=jnp.int32)

    def lane_total(v):
        for sh in (8, 4, 2, 1):
            v = v + v.at[lane_iota ^ sh].get(mode="promise_in_bounds")
        return v

    issue(0, 0)
    issue(1, 1)

    zero16 = jnp.zeros((LANES,), jnp.float32)
    rows_per_g = NB * SB
    gpf = LANES // rows_per_g

    def group(g, accs):
        accd, accq, accn = accs
        for b in range(NB):
            t = g * NB + b
            wait(b)

            @pl.when(t + 2 < NSTEP)
            def _():
                issue(t + 2, (b + 2) % NB)

            qbuf, dbuf = qbufs[b], dbufs[b]
            for r in range(SB):
                qa = tok_sum(qbuf, r * QL, QL, 10)
                da = tok_sum(dbuf, r * DL, DL, 10)
                dotv = qa[0] * da[0]
                nqv = qa[0] * qa[0]
                ndv = da[0] * da[0]
                for c in range(1, NCH):
                    dotv = dotv + qa[c] * da[c]
                    nqv = nqv + qa[c] * qa[c]
                    ndv = ndv + da[c] * da[c]
                j = (g % gpf) * rows_per_g + (b * SB + r)
                m = lane_iota == j
                accd = jnp.where(m, lane_total(dotv), accd)
                accq = jnp.where(m, lane_total(nqv), accq)
                accn = jnp.where(m, lane_total(ndv), accn)

        @pl.when(g % gpf == gpf - 1)
        def _():
            base = (g // gpf) * LANES
            dots[pl.ds(base, LANES)] = accd
            nqs[pl.ds(base, LANES)] = accq
            nds[pl.ds(base, LANES)] = accn

        flushed = (g % gpf) == gpf - 1
        accd = jnp.where(flushed, zero16, accd)
        accq = jnp.where(flushed, zero16, accq)
        accn = jnp.where(flushed, zero16, accn)
        return (accd, accq, accn)

    lax.fori_loop(0, NSTEP // NB, group, (zero16, zero16, zero16))

    def finish(i, _):
        sl = pl.ds(i * LANES, LANES)
        sq = jnp.maximum(_sqrt16(nqs[sl]), QEPS)
        sd = jnp.maximum(_sqrt16(nds[sl]), DEPS)
        sims[sl] = dots[sl] / (sq * sd)
        return 0

    lax.fori_loop(0, BW // LANES, finish, 0)
    pltpu.sync_copy(sims, out_ref.at[pl.ds(wid * BW, BW)])


@jax.jit
def _dssm(qs_r, ds_r, q_table, d_table):
    mesh = plsc.VectorSubcoreMesh(core_axis_name="c", subcore_axis_name="s")
    f = functools.partial(
        pl.kernel,
        mesh=mesh,
        compiler_params=pltpu.CompilerParams(use_tc_tiling_on_sc=False),
        out_type=jax.ShapeDtypeStruct((B,), jnp.float32),
        scratch_types=[
            pltpu.VMEM((NSTEP, QT), jnp.int32),
            pltpu.VMEM((NSTEP, DT), jnp.int32),
            pltpu.VMEM((QT, D), jnp.float32),
            pltpu.VMEM((QT, D), jnp.float32),
            pltpu.VMEM((QT, D), jnp.float32),
            pltpu.VMEM((QT, D), jnp.float32),
            pltpu.VMEM((DT, D), jnp.float32),
            pltpu.VMEM((DT, D), jnp.float32),
            pltpu.VMEM((DT, D), jnp.float32),
            pltpu.VMEM((DT, D), jnp.float32),
            pltpu.VMEM((BW,), jnp.float32),
            pltpu.VMEM((BW,), jnp.float32),
            pltpu.VMEM((BW,), jnp.float32),
            pltpu.VMEM((BW,), jnp.float32),
            pltpu.SemaphoreType.DMA,
            pltpu.SemaphoreType.DMA,
            pltpu.SemaphoreType.DMA,
            pltpu.SemaphoreType.DMA,
            pltpu.SemaphoreType.DMA,
            pltpu.SemaphoreType.DMA,
            pltpu.SemaphoreType.DMA,
            pltpu.SemaphoreType.DMA,
        ],
    )(_dssm_body)
    return f(qs_r, ds_r, q_table, d_table)


def kernel(qs, ds, rels, q_table, d_table):
    del rels
    return _dssm(qs.astype(jnp.int32), ds.astype(jnp.int32),
                 q_table, d_table)


# speedup vs baseline: 1.2648x; 1.2648x over previous
import functools

import jax
import jax.numpy as jnp
from jax import lax
from jax.experimental import pallas as pl
from jax.experimental.pallas import tpu as pltpu
from jax.experimental.pallas import tpu_sc as plsc

B = 16384
D = 64
QL = 20
DL = 50
NW = 32
BW = B // NW
SB = 2
QT = SB * QL
DT = SB * DL
NSTEP = BW // SB
NB = 4
LANES = 16
NCH = D // LANES

QEPS = float(QL * 1e-12)
DEPS = float(DL * 1e-12)


def _sqrt16(x):
    m = x
    s = jnp.full((LANES,), 1.0, jnp.float32)
    for k in (32, 16, 8, 4, 2, 1):
        c = m >= jnp.float32(4.0 ** k)
        m = jnp.where(c, m * jnp.float32(4.0 ** -k), m)
        s = jnp.where(c, s * jnp.float32(2.0 ** -k), s)
    for k in (32, 16, 8, 4, 2, 1):
        c = m < jnp.float32(4.0 ** -k)
        m = jnp.where(c, m * jnp.float32(4.0 ** k), m)
        s = jnp.where(c, s * jnp.float32(2.0 ** k), s)
    c = m >= jnp.float32(1.0)
    m = jnp.where(c, m * jnp.float32(0.25), m)
    s = jnp.where(c, s * jnp.float32(0.5), s)
    r = jnp.float32(7.0 / 3.0) - jnp.float32(4.0 / 3.0) * m
    for _ in range(5):
        r = r * (jnp.float32(1.5) - jnp.float32(0.5) * m * r * r)
    return jnp.where(x > 0, x * r * s, jnp.float32(0.0))


def _dssm_body(qs_ref, ds_ref, qt_ref, dt_ref, out_ref,
               qidx, didx, qb0, qb1, qb2, qb3, db0, db1, db2, db3,
               dots, nqs, nds, sims,
               qs0, qs1, qs2, qs3, ds0, ds1, ds2, ds3):
    wid = lax.axis_index("s") * 2 + lax.axis_index("c")
    pltpu.sync_copy(qs_ref.at[wid], qidx)
    pltpu.sync_copy(ds_ref.at[wid], didx)
    qbufs, dbufs = (qb0, qb1, qb2, qb3), (db0, db1, db2, db3)
    qsems, dsems = (qs0, qs1, qs2, qs3), (ds0, ds1, ds2, ds3)

    def issue(t, b):
        pltpu.async_copy(qt_ref.at[qidx.at[t]], qbufs[b], qsems[b])
        pltpu.async_copy(dt_ref.at[didx.at[t]], dbufs[b], dsems[b])

    def wait(b):
        pltpu.make_async_copy(qt_ref.at[pl.ds(0, QT)], qbufs[b], qsems[b]).wait()
        pltpu.make_async_copy(dt_ref.at[pl.ds(0, DT)], dbufs[b], dsems[b]).wait()

    def tok_sum(buf, base, n, unroll):
        def body(k, acc):
            return tuple(acc[c] + buf[base + k, pl.ds(c * LANES, LANES)]
                         for c in range(NCH))
        return lax.fori_loop(0, n, body,
                             tuple(jnp.zeros((LANES,), jnp.float32)
                                   for _ in range(NCH)),
                             unroll=unroll)

    lane_iota = jnp.arange(LANES, dtype=jnp.int32)

    def lane_total(v):
        for sh in (8, 4, 2, 1):
            v = v + v.at[lane_iota ^ sh].get(mode="promise_in_bounds")
        return v

    issue(0, 0)
    issue(1, 1)

    zero16 = jnp.zeros((LANES,), jnp.float32)
    rows_per_g = NB * SB
    gpf = LANES // rows_per_g

    def group(g, accs):
        accd, accq, accn = accs
        for b in range(NB):
            t = g * NB + b
            wait(b)

            @pl.when(t + 2 < NSTEP)
            def _():
                issue(t + 2, (b + 2) % NB)

            qbuf, dbuf = qbufs[b], dbufs[b]
            for r in range(SB):
                qa = tok_sum(qbuf, r * QL, QL, 10)
                da = tok_sum(dbuf, r * DL, DL, 10)
                dotv = qa[0] * da[0]
                nqv = qa[0] * qa[0]
                ndv = da[0] * da[0]
                for c in range(1, NCH):
                    dotv = dotv + qa[c] * da[c]
                    nqv = nqv + qa[c] * qa[c]
                    ndv = ndv + da[c] * da[c]
                j = (g % gpf) * rows_per_g + (b * SB + r)
                m = lane_iota == j
                accd = jnp.where(m, lane_total(dotv), accd)
                accq = jnp.where(m, lane_total(nqv), accq)
                accn = jnp.where(m, lane_total(ndv), accn)

        @pl.when(g % gpf == gpf - 1)
        def _():
            base = (g // gpf) * LANES
            dots[pl.ds(base, LANES)] = accd
            nqs[pl.ds(base, LANES)] = accq
            nds[pl.ds(base, LANES)] = accn

        flushed = (g % gpf) == gpf - 1
        accd = jnp.where(flushed, zero16, accd)
        accq = jnp.where(flushed, zero16, accq)
        accn = jnp.where(flushed, zero16, accn)
        return (accd, accq, accn)

    lax.fori_loop(0, NSTEP // NB, group, (zero16, zero16, zero16))

    def finish(i, _):
        sl = pl.ds(i * LANES, LANES)
        sq = jnp.maximum(_sqrt16(nqs[sl]), QEPS)
        sd = jnp.maximum(_sqrt16(nds[sl]), DEPS)
        sims[sl] = dots[sl] / (sq * sd)
        return 0

    lax.fori_loop(0, BW // LANES, finish, 0)
    pltpu.sync_copy(sims, out_ref.at[pl.ds(wid * BW, BW)])


@jax.jit
def _dssm(qs_r, ds_r, q_table, d_table):
    mesh = plsc.VectorSubcoreMesh(core_axis_name="c", subcore_axis_name="s")
    f = functools.partial(
        pl.kernel,
        mesh=mesh,
        compiler_params=pltpu.CompilerParams(use_tc_tiling_on_sc=False),
        out_type=jax.ShapeDtypeStruct((B,), jnp.float32),
        scratch_types=[
            pltpu.VMEM((NSTEP, QT), jnp.int32),
            pltpu.VMEM((NSTEP, DT), jnp.int32),
            pltpu.VMEM((QT, D), jnp.float32),
            pltpu.VMEM((QT, D), jnp.float32),
            pltpu.VMEM((QT, D), jnp.float32),
            pltpu.VMEM((QT, D), jnp.float32),
            pltpu.VMEM((DT, D), jnp.float32),
            pltpu.VMEM((DT, D), jnp.float32),
            pltpu.VMEM((DT, D), jnp.float32),
            pltpu.VMEM((DT, D), jnp.float32),
            pltpu.VMEM((BW,), jnp.float32),
            pltpu.VMEM((BW,), jnp.float32),
            pltpu.VMEM((BW,), jnp.float32),
            pltpu.VMEM((BW,), jnp.float32),
            pltpu.SemaphoreType.DMA,
            pltpu.SemaphoreType.DMA,
            pltpu.SemaphoreType.DMA,
            pltpu.SemaphoreType.DMA,
            pltpu.SemaphoreType.DMA,
            pltpu.SemaphoreType.DMA,
            pltpu.SemaphoreType.DMA,
            pltpu.SemaphoreType.DMA,
        ],
    )(_dssm_body)
    return f(qs_r, ds_r, q_table, d_table)


def kernel(qs, ds, rels, q_table, d_table):
    del rels
    qs_r = qs.astype(jnp.int32).reshape(NW, NSTEP, QT)
    ds_r = ds.astype(jnp.int32).reshape(NW, NSTEP, DT)
    return _dssm(qs_r, ds_r, q_table, d_table)
